# Initial kernel scaffold; baseline (speedup 1.0000x reference)
import jax
import jax.numpy as jnp
from jax.experimental import pallas as pl


def kernel(x, edge_index, edge_attr, batch, W1, b1, W2, b2, W3, b3, Wg1, as1, ad1, bg1, Wg2, as2, ad2, bg2, Wc1, bc1, Wc2, bc2):
    raise NotImplementedError("write your pallas kernel here")


# speedup vs baseline: 22.3342x; 22.3342x over previous
import functools

import jax
import jax.numpy as jnp
import numpy as np
from jax import lax
from jax.experimental import pallas as pl
from jax.experimental.pallas import tpu as pltpu
from jax.experimental.pallas import tpu_sc as plsc

N = 10000
G = 64
DF = 128
NH = 16
NP = 10240
NC = 2
NS = 16
NW = NC * NS
CB = 128
EP = 655360
NCHUNK = EP // (NW * CB)
IDXB = 16
NBLK = NCHUNK // IDXB
RPS = NP // NS

_COLPERM = np.zeros(DF, np.int32)
for _f in range(8):
    for _k in range(NH):
        _COLPERM[_f * NH + _k] = _k * 8 + _f


def _mesh():
    return plsc.VectorSubcoreMesh(core_axis_name="c", subcore_axis_name="s")


def _sc_degree(dst3, ones128, z128):
    @functools.partial(
        pl.kernel,
        out_type=jax.ShapeDtypeStruct((NC * NP, DF), jnp.float32),
        mesh=_mesh(),
        scratch_types=[
            pltpu.VMEM((IDXB, CB), jnp.int32),
            pltpu.VMEM((CB, DF), jnp.float32),
            pltpu.VMEM_SHARED((NP, DF), jnp.float32),
        ],
    )
    def deg_kernel(dst_hbm, ones_hbm, z_hbm, out_hbm, idx_v, ones_v, acc):
        c = lax.axis_index("c")
        s = lax.axis_index("s")
        w = c * NS + s
        pltpu.sync_copy(ones_hbm, ones_v)
        pltpu.sync_copy(z_hbm, acc.at[pl.ds(s * RPS, RPS)])
        plsc.subcore_barrier()

        def blk(bi, carry):
            pltpu.sync_copy(dst_hbm.at[w, pl.ds(bi * IDXB, IDXB)], idx_v)

            def body(j, cc):
                pltpu.sync_copy(ones_v, acc.at[idx_v.at[j]], add=True)
                return cc

            lax.fori_loop(0, IDXB, body, 0)
            return carry

        lax.fori_loop(0, NBLK, blk, 0)
        plsc.subcore_barrier()
        pltpu.sync_copy(acc.at[pl.ds(s * RPS, RPS)],
                        out_hbm.at[pl.ds(c * NP + s * RPS, RPS)])

    return deg_kernel(dst3, ones128, z128)


def _sc_gcn(src3, dst3, y, z128):
    @functools.partial(
        pl.kernel,
        out_type=jax.ShapeDtypeStruct((NC * NP, DF), jnp.float32),
        mesh=_mesh(),
        scratch_types=[
            pltpu.VMEM((IDXB, CB), jnp.int32),
            pltpu.VMEM((IDXB, CB), jnp.int32),
            pltpu.VMEM((CB, DF), jnp.float32),
            pltpu.VMEM_SHARED((NP, DF), jnp.float32),
            pltpu.SemaphoreType.DMA,
        ],
    )
    def gcn_kernel(src_hbm, dst_hbm, y_hbm, z_hbm, out_hbm,
                   src_v, dst_v, buf, acc, sem):
        c = lax.axis_index("c")
        s = lax.axis_index("s")
        w = c * NS + s
        pltpu.sync_copy(z_hbm, acc.at[pl.ds(s * RPS, RPS)])
        plsc.subcore_barrier()

        def blk(bi, carry):
            pltpu.sync_copy(src_hbm.at[w, pl.ds(bi * IDXB, IDXB)], src_v)
            pltpu.sync_copy(dst_hbm.at[w, pl.ds(bi * IDXB, IDXB)], dst_v)

            def body(j, cc):
                pltpu.async_copy(y_hbm.at[src_v.at[j]], buf, sem).wait()
                pltpu.sync_copy(buf, acc.at[dst_v.at[j]], add=True)
                return cc

            lax.fori_loop(0, IDXB, body, 0)
            return carry

        lax.fori_loop(0, NBLK, blk, 0)
        plsc.subcore_barrier()
        pltpu.sync_copy(acc.at[pl.ds(s * RPS, RPS)],
                        out_hbm.at[pl.ds(c * NP + s * RPS, RPS)])

    return gcn_kernel(src3, dst3, y, z128)


def _gat_edge_weight(hbuf, dbuf, aw_v, i):
    t = hbuf[i, pl.ds(0, NH)] * aw_v[0, pl.ds(0, NH)]
    for f in range(1, 8):
        t = t + hbuf[i, pl.ds(NH * f, NH)] * aw_v[f, pl.ds(0, NH)]
    for f in range(8):
        t = t + dbuf[i, pl.ds(NH * f, NH)] * aw_v[8 + f, pl.ds(0, NH)]
    e = jnp.where(t > 0.0, t, 0.2 * t)
    return jnp.exp(e)


def _sc_gat_num(src3, dst3, g, aw, z128):
    @functools.partial(
        pl.kernel,
        out_type=jax.ShapeDtypeStruct((NC * NP, DF), jnp.float32),
        mesh=_mesh(),
        scratch_types=[
            pltpu.VMEM((IDXB, CB), jnp.int32),
            pltpu.VMEM((IDXB, CB), jnp.int32),
            pltpu.VMEM((CB, DF), jnp.float32),
            pltpu.VMEM((CB, DF), jnp.float32),
            pltpu.VMEM((NH, DF), jnp.float32),
            pltpu.VMEM_SHARED((NP, DF), jnp.float32),
            pltpu.SemaphoreType.DMA,
            pltpu.SemaphoreType.DMA,
        ],
    )
    def num_kernel(src_hbm, dst_hbm, g_hbm, aw_hbm, z_hbm, out_hbm,
                   src_v, dst_v, hbuf, dbuf, aw_v, acc, sem1, sem2):
        c = lax.axis_index("c")
        s = lax.axis_index("s")
        w = c * NS + s
        pltpu.sync_copy(aw_hbm, aw_v)
        pltpu.sync_copy(z_hbm, acc.at[pl.ds(s * RPS, RPS)])
        plsc.subcore_barrier()

        def blk(bi, carry):
            pltpu.sync_copy(src_hbm.at[w, pl.ds(bi * IDXB, IDXB)], src_v)
            pltpu.sync_copy(dst_hbm.at[w, pl.ds(bi * IDXB, IDXB)], dst_v)

            def body(j, cc2):
                cp_h = pltpu.async_copy(g_hbm.at[src_v.at[j]], hbuf, sem2)
                pltpu.async_copy(g_hbm.at[dst_v.at[j]], dbuf, sem1).wait()
                cp_h.wait()

                def edge(i, cc):
                    ee = _gat_edge_weight(hbuf, dbuf, aw_v, i)
                    for f in range(8):
                        hv = hbuf[i, pl.ds(NH * f, NH)]
                        hbuf[i, pl.ds(NH * f, NH)] = hv * ee
                    return cc

                lax.fori_loop(0, CB, edge, 0)
                pltpu.sync_copy(hbuf, acc.at[dst_v.at[j]], add=True)
                return cc2

            lax.fori_loop(0, IDXB, body, 0)
            return carry

        lax.fori_loop(0, NBLK, blk, 0)
        plsc.subcore_barrier()
        pltpu.sync_copy(acc.at[pl.ds(s * RPS, RPS)],
                        out_hbm.at[pl.ds(c * NP + s * RPS, RPS)])

    return num_kernel(src3, dst3, g, aw, z128)


def _sc_gat_den(src3, dst3, g, aw, z128):
    @functools.partial(
        pl.kernel,
        out_type=jax.ShapeDtypeStruct((NC * NP, DF), jnp.float32),
        mesh=_mesh(),
        scratch_types=[
            pltpu.VMEM((IDXB, CB), jnp.int32),
            pltpu.VMEM((IDXB, CB), jnp.int32),
            pltpu.VMEM((CB, DF), jnp.float32),
            pltpu.VMEM((CB, DF), jnp.float32),
            pltpu.VMEM((NH, DF), jnp.float32),
            pltpu.VMEM_SHARED((NP, DF), jnp.float32),
            pltpu.SemaphoreType.DMA,
            pltpu.SemaphoreType.DMA,
        ],
    )
    def den_kernel(src_hbm, dst_hbm, g_hbm, aw_hbm, z_hbm, den_hbm,
                   src_v, dst_v, hbuf, dbuf, aw_v, acc, sem1, sem2):
        c = lax.axis_index("c")
        s = lax.axis_index("s")
        w = c * NS + s
        pltpu.sync_copy(aw_hbm, aw_v)
        pltpu.sync_copy(z_hbm, acc.at[pl.ds(s * RPS, RPS)])
        plsc.subcore_barrier()

        def blk(bi, carry):
            pltpu.sync_copy(src_hbm.at[w, pl.ds(bi * IDXB, IDXB)], src_v)
            pltpu.sync_copy(dst_hbm.at[w, pl.ds(bi * IDXB, IDXB)], dst_v)

            def body(j, cc2):
                cp_h = pltpu.async_copy(g_hbm.at[src_v.at[j]], hbuf, sem2)
                pltpu.async_copy(g_hbm.at[dst_v.at[j]], dbuf, sem1).wait()
                cp_h.wait()

                def edge(i, cc):
                    ee = _gat_edge_weight(hbuf, dbuf, aw_v, i)
                    for f in range(8):
                        hbuf[i, pl.ds(NH * f, NH)] = ee
                    return cc

                lax.fori_loop(0, CB, edge, 0)
                pltpu.sync_copy(hbuf, acc.at[dst_v.at[j]], add=True)
                return cc2

            lax.fori_loop(0, IDXB, body, 0)
            return carry

        lax.fori_loop(0, NBLK, blk, 0)
        plsc.subcore_barrier()
        pltpu.sync_copy(acc.at[pl.ds(s * RPS, RPS)],
                        den_hbm.at[pl.ds(c * NP + s * RPS, RPS)])

    return den_kernel(src3, dst3, g, aw, z128)


def _dot(a, b):
    return jnp.dot(a, b, preferred_element_type=jnp.float32)


def _tc_start(degp, xp, W1p):
    def body(deg_ref, x_ref, w_ref, dinv_ref, y_ref):
        dsum = deg_ref[:NP] + deg_ref[NP:]
        dinv = lax.rsqrt(jnp.maximum(dsum, 1.0))
        dinv_ref[...] = dinv
        y_ref[...] = dinv * _dot(x_ref[...], w_ref[...])

    return pl.pallas_call(
        body,
        out_shape=(jax.ShapeDtypeStruct((NP, DF), jnp.float32),
                   jax.ShapeDtypeStruct((NP, DF), jnp.float32)),
    )(degp, xp, W1p)


def _tc_gcn(sp, dinv, b, Wn):
    def body(s_ref, dinv_ref, b_ref, w_ref, y_ref):
        s = s_ref[:NP] + s_ref[NP:]
        h = jnp.maximum(dinv_ref[...] * s + b_ref[...], 0.0)
        y_ref[...] = dinv_ref[...] * _dot(h, w_ref[...])

    return pl.pallas_call(
        body,
        out_shape=jax.ShapeDtypeStruct((NP, DF), jnp.float32),
    )(sp, dinv, b, Wn)


def _tc_gat_prep1(sp, dinv, b, Wgfm):
    def body(s_ref, dinv_ref, b_ref, w_ref, g_ref):
        s = s_ref[:NP] + s_ref[NP:]
        h = jnp.maximum(dinv_ref[...] * s + b_ref[...], 0.0)
        g_ref[...] = _dot(h, w_ref[...])

    return pl.pallas_call(
        body,
        out_shape=jax.ShapeDtypeStruct((NP, DF), jnp.float32),
    )(sp, dinv, b, Wgfm)


def _tc_gat_prep2(up, denp, bgfm, Wgfm):
    def body(u_ref, den_ref, bg_ref, w_ref, g_ref):
        u = u_ref[:NP] + u_ref[NP:]
        den = den_ref[:NP] + den_ref[NP:] + 1e-16
        h = jnp.maximum(u / den + bg_ref[...], 0.0)
        g_ref[...] = _dot(h, w_ref[...])

    return pl.pallas_call(
        body,
        out_shape=jax.ShapeDtypeStruct((NP, DF), jnp.float32),
    )(up, denp, bgfm, Wgfm)


def _tc_final(up, denp, bgfm, batch2d, Wc1fm, bc1r, Wc2p, bc2p):
    def body(u_ref, den_ref, bg_ref, batch_ref, w1_ref, b1_ref,
             w2_ref, b2_ref, out_ref):
        u = u_ref[:NP] + u_ref[NP:]
        den = den_ref[:NP] + den_ref[NP:] + 1e-16
        h = u / den + bg_ref[...]
        gid = lax.broadcasted_iota(jnp.int32, (G, NP), 0)
        sel = jnp.where(gid == batch_ref[...], 1.0, 0.0)
        cnt = jnp.sum(sel, axis=1, keepdims=True)
        pooled = _dot(sel, h) / jnp.maximum(cnt, 1.0)
        hc = jnp.maximum(_dot(pooled, w1_ref[...]) + b1_ref[...], 0.0)
        out_ref[...] = _dot(hc, w2_ref[...]) + b2_ref[...]

    return pl.pallas_call(
        body,
        out_shape=jax.ShapeDtypeStruct((G, DF), jnp.float32),
    )(up, denp, bgfm, batch2d, Wc1fm, bc1r, Wc2p, bc2p)


def kernel(x, edge_index, edge_attr, batch, W1, b1, W2, b2, W3, b3,
           Wg1, as1, ad1, bg1, Wg2, as2, ad2, bg2, Wc1, bc1, Wc2, bc2):
    f32 = jnp.float32
    cp = jnp.asarray(_COLPERM)

    loops = jnp.arange(N, dtype=jnp.int32)
    src = jnp.concatenate([edge_index[0].astype(jnp.int32), loops])
    dst = jnp.concatenate([edge_index[1].astype(jnp.int32), loops])
    npad = EP - src.shape[0]
    pad = jnp.full((npad,), N, jnp.int32)
    src3 = jnp.concatenate([src, pad]).reshape(NW, NCHUNK, CB)
    dst3 = jnp.concatenate([dst, pad]).reshape(NW, NCHUNK, CB)

    xp = jnp.zeros((NP, 8), f32).at[:N, :3].set(x)
    batch2d = jnp.concatenate(
        [batch.astype(jnp.int32), jnp.full((NP - N,), -1, jnp.int32)]
    ).reshape(1, NP)

    W1p = jnp.zeros((8, DF), f32).at[:3, :].set(W1)

    Wg1fm = Wg1[:, cp]
    bg1fm = bg1[cp].reshape(1, DF)
    Wg2fm = Wg2[cp][:, cp]
    bg2fm = bg2[cp].reshape(1, DF)
    aw1 = jnp.zeros((NH, DF), f32).at[:, :NH].set(
        jnp.concatenate([as1.T, ad1.T], axis=0))
    aw2 = jnp.zeros((NH, DF), f32).at[:, :NH].set(
        jnp.concatenate([as2.T, ad2.T], axis=0))
    Wc1fm = Wc1[cp]
    bc1r = bc1.reshape(1, G)
    Wc2p = jnp.zeros((G, DF), f32).at[:, :10].set(Wc2)
    bc2p = jnp.zeros((1, DF), f32).at[0, :10].set(bc2)

    ones128 = jnp.ones((CB, DF), f32)
    z128 = jnp.zeros((RPS, DF), f32)

    degp = _sc_degree(dst3, ones128, z128)
    dinv, y1 = _tc_start(degp, xp, W1p)

    s1 = _sc_gcn(src3, dst3, y1, z128)
    y2 = _tc_gcn(s1, dinv, b1.reshape(1, DF), W2)
    s2 = _sc_gcn(src3, dst3, y2, z128)
    y3 = _tc_gcn(s2, dinv, b2.reshape(1, DF), W3)
    s3 = _sc_gcn(src3, dst3, y3, z128)

    g1 = _tc_gat_prep1(s3, dinv, b3.reshape(1, DF), Wg1fm)
    u1 = _sc_gat_num(src3, dst3, g1, aw1, z128)
    den1 = _sc_gat_den(src3, dst3, g1, aw1, z128)
    g2 = _tc_gat_prep2(u1, den1, bg1fm, Wg2fm)
    u2 = _sc_gat_num(src3, dst3, g2, aw2, z128)
    den2 = _sc_gat_den(src3, dst3, g2, aw2, z128)

    out = _tc_final(u2, den2, bg2fm, batch2d, Wc1fm, bc1r, Wc2p, bc2p)
    return out[:, :10]


# speedup vs baseline: 31.7206x; 1.4203x over previous
import functools

import jax
import jax.numpy as jnp
import numpy as np
from jax import lax
from jax.experimental import pallas as pl
from jax.experimental.pallas import tpu as pltpu
from jax.experimental.pallas import tpu_sc as plsc

N = 10000
G = 64
DF = 128
NH = 16
NP = 10240
NC = 2
NS = 16
NW = NC * NS
CB = 128
EP = 655360
NCHUNK = EP // (NW * CB)
IDXB = 16
NBLK = NCHUNK // IDXB
RPS = NP // NS

_COLPERM = np.zeros(DF, np.int32)
for _f in range(8):
    for _k in range(NH):
        _COLPERM[_f * NH + _k] = _k * 8 + _f


def _mesh():
    return plsc.VectorSubcoreMesh(core_axis_name="c", subcore_axis_name="s")


def _sc_degree(dst3, ones128, z128):
    @functools.partial(
        pl.kernel,
        out_type=jax.ShapeDtypeStruct((NC * NP, DF), jnp.float32),
        mesh=_mesh(),
        scratch_types=[
            pltpu.VMEM((IDXB, CB), jnp.int32),
            pltpu.VMEM((CB, DF), jnp.float32),
            pltpu.VMEM_SHARED((NP, DF), jnp.float32),
        ],
    )
    def deg_kernel(dst_hbm, ones_hbm, z_hbm, out_hbm, idx_v, ones_v, acc):
        c = lax.axis_index("c")
        s = lax.axis_index("s")
        w = c * NS + s
        pltpu.sync_copy(ones_hbm, ones_v)
        pltpu.sync_copy(z_hbm, acc.at[pl.ds(s * RPS, RPS)])
        plsc.subcore_barrier()

        def blk(bi, carry):
            pltpu.sync_copy(dst_hbm.at[w, pl.ds(bi * IDXB, IDXB)], idx_v)

            def body(j, cc):
                pltpu.sync_copy(ones_v, acc.at[idx_v.at[j]], add=True)
                return cc

            lax.fori_loop(0, IDXB, body, 0)
            return carry

        lax.fori_loop(0, NBLK, blk, 0)
        plsc.subcore_barrier()
        pltpu.sync_copy(acc.at[pl.ds(s * RPS, RPS)],
                        out_hbm.at[pl.ds(c * NP + s * RPS, RPS)])

    return deg_kernel(dst3, ones128, z128)


def _sc_gcn(src3, dst3, y, z128):
    @functools.partial(
        pl.kernel,
        out_type=jax.ShapeDtypeStruct((NC * NP, DF), jnp.float32),
        mesh=_mesh(),
        scratch_types=[
            pltpu.VMEM((IDXB, CB), jnp.int32),
            pltpu.VMEM((IDXB, CB), jnp.int32),
            pltpu.VMEM((CB, DF), jnp.float32),
            pltpu.VMEM_SHARED((NP, DF), jnp.float32),
            pltpu.SemaphoreType.DMA,
        ],
    )
    def gcn_kernel(src_hbm, dst_hbm, y_hbm, z_hbm, out_hbm,
                   src_v, dst_v, buf, acc, sem):
        c = lax.axis_index("c")
        s = lax.axis_index("s")
        w = c * NS + s
        pltpu.sync_copy(z_hbm, acc.at[pl.ds(s * RPS, RPS)])
        plsc.subcore_barrier()

        def blk(bi, carry):
            pltpu.sync_copy(src_hbm.at[w, pl.ds(bi * IDXB, IDXB)], src_v)
            pltpu.sync_copy(dst_hbm.at[w, pl.ds(bi * IDXB, IDXB)], dst_v)

            def body(j, cc):
                pltpu.async_copy(y_hbm.at[src_v.at[j]], buf, sem).wait()
                pltpu.sync_copy(buf, acc.at[dst_v.at[j]], add=True)
                return cc

            lax.fori_loop(0, IDXB, body, 0)
            return carry

        lax.fori_loop(0, NBLK, blk, 0)
        plsc.subcore_barrier()
        pltpu.sync_copy(acc.at[pl.ds(s * RPS, RPS)],
                        out_hbm.at[pl.ds(c * NP + s * RPS, RPS)])

    return gcn_kernel(src3, dst3, y, z128)


CBP = CB // 8
IB2 = 8
NB2 = NCHUNK // IB2


def _sc_gat_num(src3, dst3, g, aw, z128):
    @functools.partial(
        pl.kernel,
        out_type=(jax.ShapeDtypeStruct((NC * NP, DF), jnp.float32),
                  jax.ShapeDtypeStruct((NW, NCHUNK * CBP, DF), jnp.float32)),
        mesh=_mesh(),
        scratch_types=[
            pltpu.VMEM((IB2, CB), jnp.int32),
            pltpu.VMEM((IB2, CB), jnp.int32),
            pltpu.VMEM((CB, DF), jnp.float32),
            pltpu.VMEM((CB, DF), jnp.float32),
            pltpu.VMEM((CBP, DF), jnp.float32),
            pltpu.VMEM((NH, DF), jnp.float32),
            pltpu.VMEM_SHARED((NP, DF), jnp.float32),
            pltpu.SemaphoreType.DMA,
            pltpu.SemaphoreType.DMA,
        ],
    )
    def num_kernel(src_hbm, dst_hbm, g_hbm, aw_hbm, z_hbm,
                   out_hbm, ee_hbm,
                   src_v, dst_v, hbuf, dbuf, packbuf, aw_v,
                   acc, sem1, sem2):
        c = lax.axis_index("c")
        s = lax.axis_index("s")
        w = c * NS + s
        pltpu.sync_copy(aw_hbm, aw_v)
        pltpu.sync_copy(z_hbm, acc.at[pl.ds(s * RPS, RPS)])
        plsc.subcore_barrier()
        aws = [aw_v[f, pl.ds(0, NH)] for f in range(2 * 8)]

        def blk(bi, carry):
            pltpu.sync_copy(src_hbm.at[w, pl.ds(bi * IB2, IB2)], src_v)
            pltpu.sync_copy(dst_hbm.at[w, pl.ds(bi * IB2, IB2)], dst_v)

            def body(j, cc2):
                cp_h = pltpu.async_copy(g_hbm.at[src_v.at[j]], hbuf, sem2)
                pltpu.async_copy(g_hbm.at[dst_v.at[j]], dbuf, sem1).wait()
                cp_h.wait()

                def edge8(i8, cc):
                    for b in range(8):
                        i = i8 * 8 + b
                        t = hbuf[i, pl.ds(0, NH)] * aws[0]
                        for f in range(1, 8):
                            t = t + hbuf[i, pl.ds(NH * f, NH)] * aws[f]
                        for f in range(8):
                            t = t + dbuf[i, pl.ds(NH * f, NH)] * aws[8 + f]
                        e = jnp.where(t > 0.0, t, 0.2 * t)
                        ee = jnp.exp(e)
                        packbuf[i8, pl.ds(b * NH, NH)] = ee
                        for f in range(8):
                            hv = hbuf[i, pl.ds(NH * f, NH)]
                            hbuf[i, pl.ds(NH * f, NH)] = hv * ee
                    return cc

                lax.fori_loop(0, CBP, edge8, 0)
                t = bi * IB2 + j
                pltpu.sync_copy(hbuf, acc.at[dst_v.at[j]], add=True)
                pltpu.sync_copy(packbuf,
                                ee_hbm.at[w, pl.ds(t * CBP, CBP)])
                return cc2

            lax.fori_loop(0, IB2, body, 0)
            return carry

        lax.fori_loop(0, NB2, blk, 0)
        plsc.subcore_barrier()
        pltpu.sync_copy(acc.at[pl.ds(s * RPS, RPS)],
                        out_hbm.at[pl.ds(c * NP + s * RPS, RPS)])

    return num_kernel(src3, dst3, g, aw, z128)


def _sc_gat_den(dst3, eep, z128):
    @functools.partial(
        pl.kernel,
        out_type=jax.ShapeDtypeStruct((NC * NP, DF), jnp.float32),
        mesh=_mesh(),
        scratch_types=[
            pltpu.VMEM((IDXB, CB), jnp.int32),
            pltpu.VMEM((CBP, DF), jnp.float32),
            pltpu.VMEM((CB, DF), jnp.float32),
            pltpu.VMEM_SHARED((NP, DF), jnp.float32),
        ],
    )
    def den_kernel(dst_hbm, ee_hbm, z_hbm, den_hbm,
                   dst_v, pbuf, sbuf, acc):
        c = lax.axis_index("c")
        s = lax.axis_index("s")
        w = c * NS + s
        pltpu.sync_copy(z_hbm, acc.at[pl.ds(s * RPS, RPS)])
        plsc.subcore_barrier()

        def blk(bi, carry):
            pltpu.sync_copy(dst_hbm.at[w, pl.ds(bi * IDXB, IDXB)], dst_v)

            def body(j, cc2):
                t = bi * IDXB + j
                pltpu.sync_copy(ee_hbm.at[w, pl.ds(t * CBP, CBP)], pbuf)

                def unpack(i8, cc):
                    for b in range(8):
                        ee = pbuf[i8, pl.ds(b * NH, NH)]
                        for f in range(8):
                            sbuf[i8 * 8 + b, pl.ds(f * NH, NH)] = ee
                    return cc

                lax.fori_loop(0, CBP, unpack, 0)
                pltpu.sync_copy(sbuf, acc.at[dst_v.at[j]], add=True)
                return cc2

            lax.fori_loop(0, IDXB, body, 0)
            return carry

        lax.fori_loop(0, NBLK, blk, 0)
        plsc.subcore_barrier()
        pltpu.sync_copy(acc.at[pl.ds(s * RPS, RPS)],
                        den_hbm.at[pl.ds(c * NP + s * RPS, RPS)])

    return den_kernel(dst3, eep, z128)


def _dot(a, b):
    return jnp.dot(a, b, preferred_element_type=jnp.float32)


def _tc_start(degp, xp, W1p):
    def body(deg_ref, x_ref, w_ref, dinv_ref, y_ref):
        dsum = deg_ref[:NP] + deg_ref[NP:]
        dinv = lax.rsqrt(jnp.maximum(dsum, 1.0))
        dinv_ref[...] = dinv
        y_ref[...] = dinv * _dot(x_ref[...], w_ref[...])

    return pl.pallas_call(
        body,
        out_shape=(jax.ShapeDtypeStruct((NP, DF), jnp.float32),
                   jax.ShapeDtypeStruct((NP, DF), jnp.float32)),
    )(degp, xp, W1p)


def _tc_gcn(sp, dinv, b, Wn):
    def body(s_ref, dinv_ref, b_ref, w_ref, y_ref):
        s = s_ref[:NP] + s_ref[NP:]
        h = jnp.maximum(dinv_ref[...] * s + b_ref[...], 0.0)
        y_ref[...] = dinv_ref[...] * _dot(h, w_ref[...])

    return pl.pallas_call(
        body,
        out_shape=jax.ShapeDtypeStruct((NP, DF), jnp.float32),
    )(sp, dinv, b, Wn)


def _tc_gat_prep1(sp, dinv, b, Wgfm):
    def body(s_ref, dinv_ref, b_ref, w_ref, g_ref):
        s = s_ref[:NP] + s_ref[NP:]
        h = jnp.maximum(dinv_ref[...] * s + b_ref[...], 0.0)
        g_ref[...] = _dot(h, w_ref[...])

    return pl.pallas_call(
        body,
        out_shape=jax.ShapeDtypeStruct((NP, DF), jnp.float32),
    )(sp, dinv, b, Wgfm)


def _tc_gat_prep2(up, denp, bgfm, Wgfm):
    def body(u_ref, den_ref, bg_ref, w_ref, g_ref):
        u = u_ref[:NP] + u_ref[NP:]
        den = den_ref[:NP] + den_ref[NP:] + 1e-16
        h = jnp.maximum(u / den + bg_ref[...], 0.0)
        g_ref[...] = _dot(h, w_ref[...])

    return pl.pallas_call(
        body,
        out_shape=jax.ShapeDtypeStruct((NP, DF), jnp.float32),
    )(up, denp, bgfm, Wgfm)


def _tc_final(up, denp, bgfm, batch2d, Wc1fm, bc1r, Wc2p, bc2p):
    def body(u_ref, den_ref, bg_ref, batch_ref, w1_ref, b1_ref,
             w2_ref, b2_ref, out_ref):
        u = u_ref[:NP] + u_ref[NP:]
        den = den_ref[:NP] + den_ref[NP:] + 1e-16
        h = u / den + bg_ref[...]
        gid = lax.broadcasted_iota(jnp.int32, (G, NP), 0)
        sel = jnp.where(gid == batch_ref[...], 1.0, 0.0)
        cnt = jnp.sum(sel, axis=1, keepdims=True)
        pooled = _dot(sel, h) / jnp.maximum(cnt, 1.0)
        hc = jnp.maximum(_dot(pooled, w1_ref[...]) + b1_ref[...], 0.0)
        out_ref[...] = _dot(hc, w2_ref[...]) + b2_ref[...]

    return pl.pallas_call(
        body,
        out_shape=jax.ShapeDtypeStruct((G, DF), jnp.float32),
    )(up, denp, bgfm, batch2d, Wc1fm, bc1r, Wc2p, bc2p)


def kernel(x, edge_index, edge_attr, batch, W1, b1, W2, b2, W3, b3,
           Wg1, as1, ad1, bg1, Wg2, as2, ad2, bg2, Wc1, bc1, Wc2, bc2):
    f32 = jnp.float32
    cp = jnp.asarray(_COLPERM)

    loops = jnp.arange(N, dtype=jnp.int32)
    src = jnp.concatenate([edge_index[0].astype(jnp.int32), loops])
    dst = jnp.concatenate([edge_index[1].astype(jnp.int32), loops])
    npad = EP - src.shape[0]
    pad = jnp.full((npad,), N, jnp.int32)
    srcp = jnp.concatenate([src, pad])
    dstp = jnp.concatenate([dst, pad])
    src3 = srcp.reshape(NW, NCHUNK, CB)
    dst3 = dstp.reshape(NW, NCHUNK, CB)

    xp = jnp.zeros((NP, 8), f32).at[:N, :3].set(x)
    batch2d = jnp.concatenate(
        [batch.astype(jnp.int32), jnp.full((NP - N,), -1, jnp.int32)]
    ).reshape(1, NP)

    W1p = jnp.zeros((8, DF), f32).at[:3, :].set(W1)

    Wg1fm = Wg1[:, cp]
    bg1fm = bg1[cp].reshape(1, DF)
    Wg2fm = Wg2[cp][:, cp]
    bg2fm = bg2[cp].reshape(1, DF)
    aw1 = jnp.zeros((NH, DF), f32).at[:, :NH].set(
        jnp.concatenate([as1.T, ad1.T], axis=0))
    aw2 = jnp.zeros((NH, DF), f32).at[:, :NH].set(
        jnp.concatenate([as2.T, ad2.T], axis=0))
    Wc1fm = Wc1[cp]
    bc1r = bc1.reshape(1, G)
    Wc2p = jnp.zeros((G, DF), f32).at[:, :10].set(Wc2)
    bc2p = jnp.zeros((1, DF), f32).at[0, :10].set(bc2)

    ones128 = jnp.ones((CB, DF), f32)
    z128 = jnp.zeros((RPS, DF), f32)

    degp = _sc_degree(dst3, ones128, z128)
    dinv, y1 = _tc_start(degp, xp, W1p)

    s1 = _sc_gcn(src3, dst3, y1, z128)
    y2 = _tc_gcn(s1, dinv, b1.reshape(1, DF), W2)
    s2 = _sc_gcn(src3, dst3, y2, z128)
    y3 = _tc_gcn(s2, dinv, b2.reshape(1, DF), W3)
    s3 = _sc_gcn(src3, dst3, y3, z128)

    g1 = _tc_gat_prep1(s3, dinv, b3.reshape(1, DF), Wg1fm)
    u1, eep1 = _sc_gat_num(src3, dst3, g1, aw1, z128)
    den1 = _sc_gat_den(dst3, eep1, z128)
    g2 = _tc_gat_prep2(u1, den1, bg1fm, Wg2fm)
    u2, eep2 = _sc_gat_num(src3, dst3, g2, aw2, z128)
    den2 = _sc_gat_den(dst3, eep2, z128)

    out = _tc_final(u2, den2, bg2fm, batch2d, Wc1fm, bc1r, Wc2p, bc2p)
    return out[:, :10]


# speedup vs baseline: 33.5505x; 1.0577x over previous
import functools

import jax
import jax.numpy as jnp
import numpy as np
from jax import lax
from jax.experimental import pallas as pl
from jax.experimental.pallas import tpu as pltpu
from jax.experimental.pallas import tpu_sc as plsc

N = 10000
G = 64
DF = 128
NH = 16
NP = 10240
NC = 2
NS = 16
NW = NC * NS
CB = 128
EP = 655360
NCHUNK = EP // (NW * CB)
IDXB = 16
NBLK = NCHUNK // IDXB
RPS = NP // NS

_COLPERM = np.zeros(DF, np.int32)
for _f in range(8):
    for _k in range(NH):
        _COLPERM[_f * NH + _k] = _k * 8 + _f


def _mesh():
    return plsc.VectorSubcoreMesh(core_axis_name="c", subcore_axis_name="s")


def _sc_degree(dst3, ones128, z128):
    @functools.partial(
        pl.kernel,
        out_type=jax.ShapeDtypeStruct((NC * NP, DF), jnp.float32),
        mesh=_mesh(),
        scratch_types=[
            pltpu.VMEM((IDXB, CB), jnp.int32),
            pltpu.VMEM((CB, DF), jnp.float32),
            pltpu.VMEM_SHARED((NP, DF), jnp.float32),
        ],
    )
    def deg_kernel(dst_hbm, ones_hbm, z_hbm, out_hbm, idx_v, ones_v, acc):
        c = lax.axis_index("c")
        s = lax.axis_index("s")
        w = c * NS + s
        pltpu.sync_copy(ones_hbm, ones_v)
        pltpu.sync_copy(z_hbm, acc.at[pl.ds(s * RPS, RPS)])
        plsc.subcore_barrier()

        def blk(bi, carry):
            pltpu.sync_copy(dst_hbm.at[w, pl.ds(bi * IDXB, IDXB)], idx_v)

            def body(j, cc):
                pltpu.sync_copy(ones_v, acc.at[idx_v.at[j]], add=True)
                return cc

            lax.fori_loop(0, IDXB, body, 0)
            return carry

        lax.fori_loop(0, NBLK, blk, 0)
        plsc.subcore_barrier()
        pltpu.sync_copy(acc.at[pl.ds(s * RPS, RPS)],
                        out_hbm.at[pl.ds(c * NP + s * RPS, RPS)])

    return deg_kernel(dst3, ones128, z128)


def _sc_gcn(src3, dst3, y, z128):
    @functools.partial(
        pl.kernel,
        out_type=jax.ShapeDtypeStruct((NC * NP, DF), jnp.float32),
        mesh=_mesh(),
        scratch_types=[
            pltpu.VMEM((IDXB, CB), jnp.int32),
            pltpu.VMEM((IDXB, CB), jnp.int32),
            pltpu.VMEM((CB, DF), jnp.float32),
            pltpu.VMEM((CB, DF), jnp.float32),
            pltpu.VMEM_SHARED((NP, DF), jnp.float32),
            pltpu.SemaphoreType.DMA,
            pltpu.SemaphoreType.DMA,
        ],
    )
    def gcn_kernel(src_hbm, dst_hbm, y_hbm, z_hbm, out_hbm,
                   src_v, dst_v, buf0, buf1, acc, sem0, sem1):
        c = lax.axis_index("c")
        s = lax.axis_index("s")
        w = c * NS + s
        pltpu.sync_copy(z_hbm, acc.at[pl.ds(s * RPS, RPS)])
        plsc.subcore_barrier()

        def blk(bi, carry):
            pltpu.sync_copy(src_hbm.at[w, pl.ds(bi * IDXB, IDXB)], src_v)
            pltpu.sync_copy(dst_hbm.at[w, pl.ds(bi * IDXB, IDXB)], dst_v)
            pltpu.async_copy(y_hbm.at[src_v.at[0]], buf0, sem0)

            def pair(p, cc):
                j0 = p * 2
                pltpu.async_copy(y_hbm.at[src_v.at[j0 + 1]], buf1, sem1)
                pltpu.make_async_copy(y_hbm.at[src_v.at[j0]],
                                      buf0, sem0).wait()
                pltpu.sync_copy(buf0, acc.at[dst_v.at[j0]], add=True)

                @pl.when(j0 + 2 < IDXB)
                def _():
                    pltpu.async_copy(y_hbm.at[src_v.at[j0 + 2]], buf0, sem0)

                pltpu.make_async_copy(y_hbm.at[src_v.at[j0 + 1]],
                                      buf1, sem1).wait()
                pltpu.sync_copy(buf1, acc.at[dst_v.at[j0 + 1]], add=True)
                return cc

            lax.fori_loop(0, IDXB // 2, pair, 0)
            return carry

        lax.fori_loop(0, NBLK, blk, 0)
        plsc.subcore_barrier()
        pltpu.sync_copy(acc.at[pl.ds(s * RPS, RPS)],
                        out_hbm.at[pl.ds(c * NP + s * RPS, RPS)])

    return gcn_kernel(src3, dst3, y, z128)


CBP = CB // 8
IB2 = 8
NB2 = NCHUNK // IB2


def _sc_gat_num(src3, dst3, g, aw, z128):
    @functools.partial(
        pl.kernel,
        out_type=(jax.ShapeDtypeStruct((NC * NP, DF), jnp.float32),
                  jax.ShapeDtypeStruct((NW, NCHUNK * CBP, DF), jnp.float32)),
        mesh=_mesh(),
        scratch_types=[
            pltpu.VMEM((IB2, CB), jnp.int32),
            pltpu.VMEM((IB2, CB), jnp.int32),
            pltpu.VMEM((CB, DF), jnp.float32),
            pltpu.VMEM((CB, DF), jnp.float32),
            pltpu.VMEM((CBP, DF), jnp.float32),
            pltpu.VMEM((NH, DF), jnp.float32),
            pltpu.VMEM_SHARED((NP, DF), jnp.float32),
            pltpu.SemaphoreType.DMA,
            pltpu.SemaphoreType.DMA,
        ],
    )
    def num_kernel(src_hbm, dst_hbm, g_hbm, aw_hbm, z_hbm,
                   out_hbm, ee_hbm,
                   src_v, dst_v, hbuf, dbuf, packbuf, aw_v,
                   acc, sem1, sem2):
        c = lax.axis_index("c")
        s = lax.axis_index("s")
        w = c * NS + s
        pltpu.sync_copy(aw_hbm, aw_v)
        pltpu.sync_copy(z_hbm, acc.at[pl.ds(s * RPS, RPS)])
        plsc.subcore_barrier()
        aws = [aw_v[f, pl.ds(0, NH)] for f in range(2 * 8)]

        def blk(bi, carry):
            pltpu.sync_copy(src_hbm.at[w, pl.ds(bi * IB2, IB2)], src_v)
            pltpu.sync_copy(dst_hbm.at[w, pl.ds(bi * IB2, IB2)], dst_v)

            def body(j, cc2):
                cp_h = pltpu.async_copy(g_hbm.at[src_v.at[j]], hbuf, sem2)
                pltpu.async_copy(g_hbm.at[dst_v.at[j]], dbuf, sem1).wait()
                cp_h.wait()

                def edge8(i8, cc):
                    for b in range(8):
                        i = i8 * 8 + b
                        t = hbuf[i, pl.ds(0, NH)] * aws[0]
                        for f in range(1, 8):
                            t = t + hbuf[i, pl.ds(NH * f, NH)] * aws[f]
                        for f in range(8):
                            t = t + dbuf[i, pl.ds(NH * f, NH)] * aws[8 + f]
                        e = jnp.where(t > 0.0, t, 0.2 * t)
                        ee = jnp.exp(e)
                        packbuf[i8, pl.ds(b * NH, NH)] = ee
                        for f in range(8):
                            hv = hbuf[i, pl.ds(NH * f, NH)]
                            hbuf[i, pl.ds(NH * f, NH)] = hv * ee
                    return cc

                lax.fori_loop(0, CBP, edge8, 0)
                t = bi * IB2 + j
                pltpu.sync_copy(hbuf, acc.at[dst_v.at[j]], add=True)
                pltpu.sync_copy(packbuf,
                                ee_hbm.at[w, pl.ds(t * CBP, CBP)])
                return cc2

            lax.fori_loop(0, IB2, body, 0)
            return carry

        lax.fori_loop(0, NB2, blk, 0)
        plsc.subcore_barrier()
        pltpu.sync_copy(acc.at[pl.ds(s * RPS, RPS)],
                        out_hbm.at[pl.ds(c * NP + s * RPS, RPS)])

    return num_kernel(src3, dst3, g, aw, z128)


def _sc_gat_den(dst3, eep, z128):
    @functools.partial(
        pl.kernel,
        out_type=jax.ShapeDtypeStruct((NC * NP, DF), jnp.float32),
        mesh=_mesh(),
        scratch_types=[
            pltpu.VMEM((IDXB, CB), jnp.int32),
            pltpu.VMEM((CBP, DF), jnp.float32),
            pltpu.VMEM((CB, DF), jnp.float32),
            pltpu.VMEM_SHARED((NP, DF), jnp.float32),
        ],
    )
    def den_kernel(dst_hbm, ee_hbm, z_hbm, den_hbm,
                   dst_v, pbuf, sbuf, acc):
        c = lax.axis_index("c")
        s = lax.axis_index("s")
        w = c * NS + s
        pltpu.sync_copy(z_hbm, acc.at[pl.ds(s * RPS, RPS)])
        plsc.subcore_barrier()

        def blk(bi, carry):
            pltpu.sync_copy(dst_hbm.at[w, pl.ds(bi * IDXB, IDXB)], dst_v)

            def body(j, cc2):
                t = bi * IDXB + j
                pltpu.sync_copy(ee_hbm.at[w, pl.ds(t * CBP, CBP)], pbuf)

                def unpack(i8, cc):
                    for b in range(8):
                        ee = pbuf[i8, pl.ds(b * NH, NH)]
                        for f in range(8):
                            sbuf[i8 * 8 + b, pl.ds(f * NH, NH)] = ee
                    return cc

                lax.fori_loop(0, CBP, unpack, 0)
                pltpu.sync_copy(sbuf, acc.at[dst_v.at[j]], add=True)
                return cc2

            lax.fori_loop(0, IDXB, body, 0)
            return carry

        lax.fori_loop(0, NBLK, blk, 0)
        plsc.subcore_barrier()
        pltpu.sync_copy(acc.at[pl.ds(s * RPS, RPS)],
                        den_hbm.at[pl.ds(c * NP + s * RPS, RPS)])

    return den_kernel(dst3, eep, z128)


def _dot(a, b):
    return jnp.dot(a, b, preferred_element_type=jnp.float32)


def _tc_start(degp, xp, W1p):
    def body(deg_ref, x_ref, w_ref, dinv_ref, y_ref):
        dsum = deg_ref[:NP] + deg_ref[NP:]
        dinv = lax.rsqrt(jnp.maximum(dsum, 1.0))
        dinv_ref[...] = dinv
        y_ref[...] = dinv * _dot(x_ref[...], w_ref[...])

    return pl.pallas_call(
        body,
        out_shape=(jax.ShapeDtypeStruct((NP, DF), jnp.float32),
                   jax.ShapeDtypeStruct((NP, DF), jnp.float32)),
    )(degp, xp, W1p)


def _tc_gcn(sp, dinv, b, Wn):
    def body(s_ref, dinv_ref, b_ref, w_ref, y_ref):
        s = s_ref[:NP] + s_ref[NP:]
        h = jnp.maximum(dinv_ref[...] * s + b_ref[...], 0.0)
        y_ref[...] = dinv_ref[...] * _dot(h, w_ref[...])

    return pl.pallas_call(
        body,
        out_shape=jax.ShapeDtypeStruct((NP, DF), jnp.float32),
    )(sp, dinv, b, Wn)


def _tc_gat_prep1(sp, dinv, b, Wgfm):
    def body(s_ref, dinv_ref, b_ref, w_ref, g_ref):
        s = s_ref[:NP] + s_ref[NP:]
        h = jnp.maximum(dinv_ref[...] * s + b_ref[...], 0.0)
        g_ref[...] = _dot(h, w_ref[...])

    return pl.pallas_call(
        body,
        out_shape=jax.ShapeDtypeStruct((NP, DF), jnp.float32),
    )(sp, dinv, b, Wgfm)


def _tc_gat_prep2(up, denp, bgfm, Wgfm):
    def body(u_ref, den_ref, bg_ref, w_ref, g_ref):
        u = u_ref[:NP] + u_ref[NP:]
        den = den_ref[:NP] + den_ref[NP:] + 1e-16
        h = jnp.maximum(u / den + bg_ref[...], 0.0)
        g_ref[...] = _dot(h, w_ref[...])

    return pl.pallas_call(
        body,
        out_shape=jax.ShapeDtypeStruct((NP, DF), jnp.float32),
    )(up, denp, bgfm, Wgfm)


def _tc_final(up, denp, bgfm, batch2d, Wc1fm, bc1r, Wc2p, bc2p):
    def body(u_ref, den_ref, bg_ref, batch_ref, w1_ref, b1_ref,
             w2_ref, b2_ref, out_ref):
        u = u_ref[:NP] + u_ref[NP:]
        den = den_ref[:NP] + den_ref[NP:] + 1e-16
        h = u / den + bg_ref[...]
        gid = lax.broadcasted_iota(jnp.int32, (G, NP), 0)
        sel = jnp.where(gid == batch_ref[...], 1.0, 0.0)
        cnt = jnp.sum(sel, axis=1, keepdims=True)
        pooled = _dot(sel, h) / jnp.maximum(cnt, 1.0)
        hc = jnp.maximum(_dot(pooled, w1_ref[...]) + b1_ref[...], 0.0)
        out_ref[...] = _dot(hc, w2_ref[...]) + b2_ref[...]

    return pl.pallas_call(
        body,
        out_shape=jax.ShapeDtypeStruct((G, DF), jnp.float32),
    )(up, denp, bgfm, batch2d, Wc1fm, bc1r, Wc2p, bc2p)


def kernel(x, edge_index, edge_attr, batch, W1, b1, W2, b2, W3, b3,
           Wg1, as1, ad1, bg1, Wg2, as2, ad2, bg2, Wc1, bc1, Wc2, bc2):
    f32 = jnp.float32
    cp = jnp.asarray(_COLPERM)

    loops = jnp.arange(N, dtype=jnp.int32)
    src = jnp.concatenate([edge_index[0].astype(jnp.int32), loops])
    dst = jnp.concatenate([edge_index[1].astype(jnp.int32), loops])
    npad = EP - src.shape[0]
    pad = jnp.full((npad,), N, jnp.int32)
    srcp = jnp.concatenate([src, pad])
    dstp = jnp.concatenate([dst, pad])
    src3 = srcp.reshape(NW, NCHUNK, CB)
    dst3 = dstp.reshape(NW, NCHUNK, CB)

    xp = jnp.zeros((NP, 8), f32).at[:N, :3].set(x)
    batch2d = jnp.concatenate(
        [batch.astype(jnp.int32), jnp.full((NP - N,), -1, jnp.int32)]
    ).reshape(1, NP)

    W1p = jnp.zeros((8, DF), f32).at[:3, :].set(W1)

    Wg1fm = Wg1[:, cp]
    bg1fm = bg1[cp].reshape(1, DF)
    Wg2fm = Wg2[cp][:, cp]
    bg2fm = bg2[cp].reshape(1, DF)
    aw1 = jnp.zeros((NH, DF), f32).at[:, :NH].set(
        jnp.concatenate([as1.T, ad1.T], axis=0))
    aw2 = jnp.zeros((NH, DF), f32).at[:, :NH].set(
        jnp.concatenate([as2.T, ad2.T], axis=0))
    Wc1fm = Wc1[cp]
    bc1r = bc1.reshape(1, G)
    Wc2p = jnp.zeros((G, DF), f32).at[:, :10].set(Wc2)
    bc2p = jnp.zeros((1, DF), f32).at[0, :10].set(bc2)

    ones128 = jnp.ones((CB, DF), f32)
    z128 = jnp.zeros((RPS, DF), f32)

    degp = _sc_degree(dst3, ones128, z128)
    dinv, y1 = _tc_start(degp, xp, W1p)

    s1 = _sc_gcn(src3, dst3, y1, z128)
    y2 = _tc_gcn(s1, dinv, b1.reshape(1, DF), W2)
    s2 = _sc_gcn(src3, dst3, y2, z128)
    y3 = _tc_gcn(s2, dinv, b2.reshape(1, DF), W3)
    s3 = _sc_gcn(src3, dst3, y3, z128)

    g1 = _tc_gat_prep1(s3, dinv, b3.reshape(1, DF), Wg1fm)
    u1, eep1 = _sc_gat_num(src3, dst3, g1, aw1, z128)
    den1 = _sc_gat_den(dst3, eep1, z128)
    g2 = _tc_gat_prep2(u1, den1, bg1fm, Wg2fm)
    u2, eep2 = _sc_gat_num(src3, dst3, g2, aw2, z128)
    den2 = _sc_gat_den(dst3, eep2, z128)

    out = _tc_final(u2, den2, bg2fm, batch2d, Wc1fm, bc1r, Wc2p, bc2p)
    return out[:, :10]


# speedup vs baseline: 34.6293x; 1.0322x over previous
import functools

import jax
import jax.numpy as jnp
import numpy as np
from jax import lax
from jax.experimental import pallas as pl
from jax.experimental.pallas import tpu as pltpu
from jax.experimental.pallas import tpu_sc as plsc

N = 10000
G = 64
DF = 128
NH = 16
NP = 10240
NC = 2
NS = 16
NW = NC * NS
CB = 128
EP = 655360
NCHUNK = EP // (NW * CB)
IDXB = 16
NBLK = NCHUNK // IDXB
RPS = NP // NS

_COLPERM = np.zeros(DF, np.int32)
for _f in range(8):
    for _k in range(NH):
        _COLPERM[_f * NH + _k] = _k * 8 + _f


def _mesh():
    return plsc.VectorSubcoreMesh(core_axis_name="c", subcore_axis_name="s")


def _sc_degree(dst3, ones128, z128):
    @functools.partial(
        pl.kernel,
        out_type=jax.ShapeDtypeStruct((NC * NP, DF), jnp.float32),
        mesh=_mesh(),
        scratch_types=[
            pltpu.VMEM((IDXB, CB), jnp.int32),
            pltpu.VMEM((CB, DF), jnp.float32),
            pltpu.VMEM_SHARED((NP, DF), jnp.float32),
        ],
    )
    def deg_kernel(dst_hbm, ones_hbm, z_hbm, out_hbm, idx_v, ones_v, acc):
        c = lax.axis_index("c")
        s = lax.axis_index("s")
        w = c * NS + s
        pltpu.sync_copy(ones_hbm, ones_v)
        pltpu.sync_copy(z_hbm, acc.at[pl.ds(s * RPS, RPS)])
        plsc.subcore_barrier()

        def blk(bi, carry):
            pltpu.sync_copy(dst_hbm.at[w, pl.ds(bi * IDXB, IDXB)], idx_v)

            def body(j, cc):
                pltpu.sync_copy(ones_v, acc.at[idx_v.at[j]], add=True)
                return cc

            lax.fori_loop(0, IDXB, body, 0)
            return carry

        lax.fori_loop(0, NBLK, blk, 0)
        plsc.subcore_barrier()
        pltpu.sync_copy(acc.at[pl.ds(s * RPS, RPS)],
                        out_hbm.at[pl.ds(c * NP + s * RPS, RPS)])

    return deg_kernel(dst3, ones128, z128)


def _sc_gcn(src3, dst3, y, z128):
    @functools.partial(
        pl.kernel,
        out_type=jax.ShapeDtypeStruct((NC * NP, DF), jnp.float32),
        mesh=_mesh(),
        scratch_types=[
            pltpu.VMEM((IDXB, CB), jnp.int32),
            pltpu.VMEM((IDXB, CB), jnp.int32),
            pltpu.VMEM((CB, DF), jnp.float32),
            pltpu.VMEM((CB, DF), jnp.float32),
            pltpu.VMEM_SHARED((NP, DF), jnp.float32),
            pltpu.SemaphoreType.DMA,
            pltpu.SemaphoreType.DMA,
        ],
    )
    def gcn_kernel(src_hbm, dst_hbm, y_hbm, z_hbm, out_hbm,
                   src_v, dst_v, buf0, buf1, acc, sem0, sem1):
        c = lax.axis_index("c")
        s = lax.axis_index("s")
        w = c * NS + s
        pltpu.sync_copy(z_hbm, acc.at[pl.ds(s * RPS, RPS)])
        plsc.subcore_barrier()

        def blk(bi, carry):
            pltpu.sync_copy(src_hbm.at[w, pl.ds(bi * IDXB, IDXB)], src_v)
            pltpu.sync_copy(dst_hbm.at[w, pl.ds(bi * IDXB, IDXB)], dst_v)
            pltpu.async_copy(y_hbm.at[src_v.at[0]], buf0, sem0)

            def pair(p, cc):
                j0 = p * 2
                pltpu.async_copy(y_hbm.at[src_v.at[j0 + 1]], buf1, sem1)
                pltpu.make_async_copy(y_hbm.at[src_v.at[j0]],
                                      buf0, sem0).wait()
                pltpu.sync_copy(buf0, acc.at[dst_v.at[j0]], add=True)

                @pl.when(j0 + 2 < IDXB)
                def _():
                    pltpu.async_copy(y_hbm.at[src_v.at[j0 + 2]], buf0, sem0)

                pltpu.make_async_copy(y_hbm.at[src_v.at[j0 + 1]],
                                      buf1, sem1).wait()
                pltpu.sync_copy(buf1, acc.at[dst_v.at[j0 + 1]], add=True)
                return cc

            lax.fori_loop(0, IDXB // 2, pair, 0)
            return carry

        lax.fori_loop(0, NBLK, blk, 0)
        plsc.subcore_barrier()
        pltpu.sync_copy(acc.at[pl.ds(s * RPS, RPS)],
                        out_hbm.at[pl.ds(c * NP + s * RPS, RPS)])

    return gcn_kernel(src3, dst3, y, z128)


CBP = CB // 8
IB2 = 8
NB2 = NCHUNK // IB2


def _sc_gat_num(src3, dst3, g, aw, z128):
    @functools.partial(
        pl.kernel,
        out_type=(jax.ShapeDtypeStruct((NC * NP, DF), jnp.float32),
                  jax.ShapeDtypeStruct((NW, NCHUNK * CBP, DF), jnp.float32)),
        mesh=_mesh(),
        scratch_types=[
            pltpu.VMEM((IB2, CB), jnp.int32),
            pltpu.VMEM((IB2, CB), jnp.int32),
            pltpu.VMEM((CB, DF), jnp.float32),
            pltpu.VMEM((CB, DF), jnp.float32),
            pltpu.VMEM((CBP, DF), jnp.float32),
            pltpu.VMEM((NH, DF), jnp.float32),
            pltpu.VMEM_SHARED((NP, DF), jnp.float32),
            pltpu.SemaphoreType.DMA,
            pltpu.SemaphoreType.DMA,
        ],
    )
    def num_kernel(src_hbm, dst_hbm, g_hbm, aw_hbm, z_hbm,
                   out_hbm, ee_hbm,
                   src_v, dst_v, hbuf, dbuf, packbuf, aw_v,
                   acc, sem1, sem2):
        c = lax.axis_index("c")
        s = lax.axis_index("s")
        w = c * NS + s
        pltpu.sync_copy(aw_hbm, aw_v)
        pltpu.sync_copy(z_hbm, acc.at[pl.ds(s * RPS, RPS)])
        plsc.subcore_barrier()
        aws = [aw_v[f, pl.ds(0, NH)] for f in range(2 * 8)]

        def blk(bi, carry):
            pltpu.sync_copy(src_hbm.at[w, pl.ds(bi * IB2, IB2)], src_v)
            pltpu.sync_copy(dst_hbm.at[w, pl.ds(bi * IB2, IB2)], dst_v)

            def body(j, cc2):
                cp_h = pltpu.async_copy(g_hbm.at[src_v.at[j]], hbuf, sem2)
                pltpu.async_copy(g_hbm.at[dst_v.at[j]], dbuf, sem1).wait()
                cp_h.wait()

                @plsc.parallel_loop(0, CBP, unroll=2)
                def edge8(i8):
                    for b in range(8):
                        i = i8 * 8 + b
                        t = hbuf[i, pl.ds(0, NH)] * aws[0]
                        for f in range(1, 8):
                            t = t + hbuf[i, pl.ds(NH * f, NH)] * aws[f]
                        for f in range(8):
                            t = t + dbuf[i, pl.ds(NH * f, NH)] * aws[8 + f]
                        e = jnp.where(t > 0.0, t, 0.2 * t)
                        ee = jnp.exp(e)
                        packbuf[i8, pl.ds(b * NH, NH)] = ee
                        for f in range(8):
                            hv = hbuf[i, pl.ds(NH * f, NH)]
                            hbuf[i, pl.ds(NH * f, NH)] = hv * ee

                t = bi * IB2 + j
                pltpu.sync_copy(hbuf, acc.at[dst_v.at[j]], add=True)
                pltpu.sync_copy(packbuf,
                                ee_hbm.at[w, pl.ds(t * CBP, CBP)])
                return cc2

            lax.fori_loop(0, IB2, body, 0)
            return carry

        lax.fori_loop(0, NB2, blk, 0)
        plsc.subcore_barrier()
        pltpu.sync_copy(acc.at[pl.ds(s * RPS, RPS)],
                        out_hbm.at[pl.ds(c * NP + s * RPS, RPS)])

    return num_kernel(src3, dst3, g, aw, z128)


def _sc_gat_den(dst3, eep, z128):
    @functools.partial(
        pl.kernel,
        out_type=jax.ShapeDtypeStruct((NC * NP, DF), jnp.float32),
        mesh=_mesh(),
        scratch_types=[
            pltpu.VMEM((IDXB, CB), jnp.int32),
            pltpu.VMEM((CBP, DF), jnp.float32),
            pltpu.VMEM((CB, DF), jnp.float32),
            pltpu.VMEM_SHARED((NP, DF), jnp.float32),
        ],
    )
    def den_kernel(dst_hbm, ee_hbm, z_hbm, den_hbm,
                   dst_v, pbuf, sbuf, acc):
        c = lax.axis_index("c")
        s = lax.axis_index("s")
        w = c * NS + s
        pltpu.sync_copy(z_hbm, acc.at[pl.ds(s * RPS, RPS)])
        plsc.subcore_barrier()

        def blk(bi, carry):
            pltpu.sync_copy(dst_hbm.at[w, pl.ds(bi * IDXB, IDXB)], dst_v)

            def body(j, cc2):
                t = bi * IDXB + j
                pltpu.sync_copy(ee_hbm.at[w, pl.ds(t * CBP, CBP)], pbuf)

                @plsc.parallel_loop(0, CBP, unroll=2)
                def unpack(i8):
                    for b in range(8):
                        ee = pbuf[i8, pl.ds(b * NH, NH)]
                        for f in range(8):
                            sbuf[i8 * 8 + b, pl.ds(f * NH, NH)] = ee
                pltpu.sync_copy(sbuf, acc.at[dst_v.at[j]], add=True)
                return cc2

            lax.fori_loop(0, IDXB, body, 0)
            return carry

        lax.fori_loop(0, NBLK, blk, 0)
        plsc.subcore_barrier()
        pltpu.sync_copy(acc.at[pl.ds(s * RPS, RPS)],
                        den_hbm.at[pl.ds(c * NP + s * RPS, RPS)])

    return den_kernel(dst3, eep, z128)


def _dot(a, b):
    return jnp.dot(a, b, preferred_element_type=jnp.float32)


def _tc_start(degp, xp, W1p):
    def body(deg_ref, x_ref, w_ref, dinv_ref, y_ref):
        dsum = deg_ref[:NP] + deg_ref[NP:]
        dinv = lax.rsqrt(jnp.maximum(dsum, 1.0))
        dinv_ref[...] = dinv
        y_ref[...] = dinv * _dot(x_ref[...], w_ref[...])

    return pl.pallas_call(
        body,
        out_shape=(jax.ShapeDtypeStruct((NP, DF), jnp.float32),
                   jax.ShapeDtypeStruct((NP, DF), jnp.float32)),
    )(degp, xp, W1p)


def _tc_gcn(sp, dinv, b, Wn):
    def body(s_ref, dinv_ref, b_ref, w_ref, y_ref):
        s = s_ref[:NP] + s_ref[NP:]
        h = jnp.maximum(dinv_ref[...] * s + b_ref[...], 0.0)
        y_ref[...] = dinv_ref[...] * _dot(h, w_ref[...])

    return pl.pallas_call(
        body,
        out_shape=jax.ShapeDtypeStruct((NP, DF), jnp.float32),
    )(sp, dinv, b, Wn)


def _tc_gat_prep1(sp, dinv, b, Wgfm):
    def body(s_ref, dinv_ref, b_ref, w_ref, g_ref):
        s = s_ref[:NP] + s_ref[NP:]
        h = jnp.maximum(dinv_ref[...] * s + b_ref[...], 0.0)
        g_ref[...] = _dot(h, w_ref[...])

    return pl.pallas_call(
        body,
        out_shape=jax.ShapeDtypeStruct((NP, DF), jnp.float32),
    )(sp, dinv, b, Wgfm)


def _tc_gat_prep2(up, denp, bgfm, Wgfm):
    def body(u_ref, den_ref, bg_ref, w_ref, g_ref):
        u = u_ref[:NP] + u_ref[NP:]
        den = den_ref[:NP] + den_ref[NP:] + 1e-16
        h = jnp.maximum(u / den + bg_ref[...], 0.0)
        g_ref[...] = _dot(h, w_ref[...])

    return pl.pallas_call(
        body,
        out_shape=jax.ShapeDtypeStruct((NP, DF), jnp.float32),
    )(up, denp, bgfm, Wgfm)


def _tc_final(up, denp, bgfm, batch2d, Wc1fm, bc1r, Wc2p, bc2p):
    def body(u_ref, den_ref, bg_ref, batch_ref, w1_ref, b1_ref,
             w2_ref, b2_ref, out_ref):
        u = u_ref[:NP] + u_ref[NP:]
        den = den_ref[:NP] + den_ref[NP:] + 1e-16
        h = u / den + bg_ref[...]
        gid = lax.broadcasted_iota(jnp.int32, (G, NP), 0)
        sel = jnp.where(gid == batch_ref[...], 1.0, 0.0)
        cnt = jnp.sum(sel, axis=1, keepdims=True)
        pooled = _dot(sel, h) / jnp.maximum(cnt, 1.0)
        hc = jnp.maximum(_dot(pooled, w1_ref[...]) + b1_ref[...], 0.0)
        out_ref[...] = _dot(hc, w2_ref[...]) + b2_ref[...]

    return pl.pallas_call(
        body,
        out_shape=jax.ShapeDtypeStruct((G, DF), jnp.float32),
    )(up, denp, bgfm, batch2d, Wc1fm, bc1r, Wc2p, bc2p)


def kernel(x, edge_index, edge_attr, batch, W1, b1, W2, b2, W3, b3,
           Wg1, as1, ad1, bg1, Wg2, as2, ad2, bg2, Wc1, bc1, Wc2, bc2):
    f32 = jnp.float32
    cp = jnp.asarray(_COLPERM)

    loops = jnp.arange(N, dtype=jnp.int32)
    src = jnp.concatenate([edge_index[0].astype(jnp.int32), loops])
    dst = jnp.concatenate([edge_index[1].astype(jnp.int32), loops])
    npad = EP - src.shape[0]
    pad = jnp.full((npad,), N, jnp.int32)
    srcp = jnp.concatenate([src, pad])
    dstp = jnp.concatenate([dst, pad])
    src3 = srcp.reshape(NW, NCHUNK, CB)
    dst3 = dstp.reshape(NW, NCHUNK, CB)

    xp = jnp.zeros((NP, 8), f32).at[:N, :3].set(x)
    batch2d = jnp.concatenate(
        [batch.astype(jnp.int32), jnp.full((NP - N,), -1, jnp.int32)]
    ).reshape(1, NP)

    W1p = jnp.zeros((8, DF), f32).at[:3, :].set(W1)

    Wg1fm = Wg1[:, cp]
    bg1fm = bg1[cp].reshape(1, DF)
    Wg2fm = Wg2[cp][:, cp]
    bg2fm = bg2[cp].reshape(1, DF)
    aw1 = jnp.zeros((NH, DF), f32).at[:, :NH].set(
        jnp.concatenate([as1.T, ad1.T], axis=0))
    aw2 = jnp.zeros((NH, DF), f32).at[:, :NH].set(
        jnp.concatenate([as2.T, ad2.T], axis=0))
    Wc1fm = Wc1[cp]
    bc1r = bc1.reshape(1, G)
    Wc2p = jnp.zeros((G, DF), f32).at[:, :10].set(Wc2)
    bc2p = jnp.zeros((1, DF), f32).at[0, :10].set(bc2)

    ones128 = jnp.ones((CB, DF), f32)
    z128 = jnp.zeros((RPS, DF), f32)

    degp = _sc_degree(dst3, ones128, z128)
    dinv, y1 = _tc_start(degp, xp, W1p)

    s1 = _sc_gcn(src3, dst3, y1, z128)
    y2 = _tc_gcn(s1, dinv, b1.reshape(1, DF), W2)
    s2 = _sc_gcn(src3, dst3, y2, z128)
    y3 = _tc_gcn(s2, dinv, b2.reshape(1, DF), W3)
    s3 = _sc_gcn(src3, dst3, y3, z128)

    g1 = _tc_gat_prep1(s3, dinv, b3.reshape(1, DF), Wg1fm)
    u1, eep1 = _sc_gat_num(src3, dst3, g1, aw1, z128)
    den1 = _sc_gat_den(dst3, eep1, z128)
    g2 = _tc_gat_prep2(u1, den1, bg1fm, Wg2fm)
    u2, eep2 = _sc_gat_num(src3, dst3, g2, aw2, z128)
    den2 = _sc_gat_den(dst3, eep2, z128)

    out = _tc_final(u2, den2, bg2fm, batch2d, Wc1fm, bc1r, Wc2p, bc2p)
    return out[:, :10]
